# baseline (speedup 1.0000x reference)
import functools

import jax
import jax.numpy as jnp
import numpy as np
from jax.experimental import pallas as pl
from jax.experimental.pallas import tpu as pltpu

_GRID_RANGE = ((-3.0, 3.0), (-3.0, 3.0))
_POINTS = (48, 48)
_TM = 576


def _axes():
    return [np.linspace(lo, hi, p, dtype=np.float32)
            for (lo, hi), p in zip(_GRID_RANGE, _POINTS)]


_GRID = np.stack(np.meshgrid(*_axes(), indexing="ij"), axis=-1)
_GXY = np.stack(_axes(), axis=-1)


def _setconv_kernel(gxy_ref, ls_ref, xst_ref, z_ref, out_ref,
                    w_ref, obuf_ref, sem_ref):
    bb = pl.program_id(0)
    nb = pl.num_programs(0)
    inv = 1.0 / (1e-05 + jax.nn.softplus(ls_ref[...]))
    gs = gxy_ref[...] * inv
    gx = gs[:, 0:1]
    gy = gs[:, 1:2]
    xs0 = xst_ref[0, 0:1, :] * inv[:, 0:1]
    xs1 = xst_ref[0, 1:2, :] * inv[:, 1:2]
    dx = gx - xs0
    dy = gy - xs1
    ex = jnp.exp(-0.5 * dx * dx)
    ey = jnp.exp(-0.5 * dy * dy)
    z = z_ref[0]
    n_iy = ey.shape[0]
    rows_per_tile = _TM // n_iy
    n_tiles = obuf_ref.shape[0]

    def _copy(g):
        return pltpu.make_async_copy(
            obuf_ref.at[g],
            out_ref.at[bb, pl.ds(g * _TM, _TM), :],
            sem_ref.at[g])

    for g in range(n_tiles):
        @pl.when(bb > 0)
        def _():
            _copy(g).wait()
        for j in range(rows_per_tile):
            ix = g * rows_per_tile + j
            w_ref[j * n_iy:(j + 1) * n_iy, :] = ex[ix:ix + 1, :] * ey
        obuf_ref[g] = jnp.dot(w_ref[...], z,
                              preferred_element_type=jnp.float32)
        _copy(g).start()

    @pl.when(bb == nb - 1)
    def _():
        for g in range(n_tiles):
            _copy(g).wait()


@functools.partial(jax.jit, static_argnames=("m",))
def _run(gxy, ls, xst, z, m):
    b, _, n = xst.shape
    dz = z.shape[-1]
    n_tiles = m // _TM
    return pl.pallas_call(
        _setconv_kernel,
        grid=(b,),
        in_specs=[
            pl.BlockSpec((48, 2), lambda bb: (0, 0)),
            pl.BlockSpec((1, 2), lambda bb: (0, 0)),
            pl.BlockSpec((1, 2, n), lambda bb: (bb, 0, 0)),
            pl.BlockSpec((1, n, dz), lambda bb: (bb, 0, 0)),
        ],
        out_specs=pl.BlockSpec(memory_space=pltpu.MemorySpace.HBM),
        out_shape=jax.ShapeDtypeStruct((b, m, dz), jnp.float32),
        scratch_shapes=[
            pltpu.VMEM((_TM, n), jnp.float32),
            pltpu.VMEM((n_tiles, _TM, dz), jnp.float32),
            pltpu.SemaphoreType.DMA((n_tiles,)),
        ],
    )(gxy, ls, xst, z)


def kernel(x, z, lengthscale_param):
    b, n, dx = x.shape
    dz = z.shape[-1]
    m = _POINTS[0] * _POINTS[1]
    gxy = jnp.asarray(_GXY)
    ls = lengthscale_param.reshape(1, 2)
    xst = x.transpose(0, 2, 1)
    z_grid_flat = _run(gxy, ls, xst, z, m=m)
    z_grid = z_grid_flat.reshape((b,) + _POINTS + (dz,))
    x_grid = jnp.broadcast_to(jnp.asarray(_GRID)[None], (b,) + _GRID.shape)
    return (x_grid, z_grid)


# speedup vs baseline: 1.8343x; 1.8343x over previous
import functools

import jax
import jax.numpy as jnp
import numpy as np
from jax.experimental import pallas as pl
from jax.experimental.pallas import tpu as pltpu

_GRID_RANGE = ((-3.0, 3.0), (-3.0, 3.0))
_POINTS = (48, 48)
_TM = 576
_NB = 4


def _axes():
    return [np.linspace(lo, hi, p, dtype=np.float32)
            for (lo, hi), p in zip(_GRID_RANGE, _POINTS)]


_GRID = np.stack(np.meshgrid(*_axes(), indexing="ij"), axis=-1)
_GXY = np.stack(_axes(), axis=-1)


def _setconv_kernel(gxy_ref, ls_ref, xst_ref, z_ref, out_ref, w_ref):
    inv = 1.0 / (1e-05 + jax.nn.softplus(ls_ref[...]))
    gs = gxy_ref[...] * inv
    gx = gs[:, 0:1]
    gy = gs[:, 1:2]
    n_iy = gy.shape[0]
    rows_per_tile = _TM // n_iy
    n_tiles = (gx.shape[0] * n_iy) // _TM
    for bi in range(xst_ref.shape[0]):
        xs0 = xst_ref[bi, 0:1, :] * inv[:, 0:1]
        xs1 = xst_ref[bi, 1:2, :] * inv[:, 1:2]
        dx = gx - xs0
        dy = gy - xs1
        ex = jnp.exp(-0.5 * dx * dx)
        ey = jnp.exp(-0.5 * dy * dy)
        z = z_ref[bi]
        for g in range(n_tiles):
            for j in range(rows_per_tile):
                ix = g * rows_per_tile + j
                w_ref[j * n_iy:(j + 1) * n_iy, :] = ex[ix:ix + 1, :] * ey
            out_ref[bi, g * _TM:(g + 1) * _TM, :] = jnp.dot(
                w_ref[...], z, preferred_element_type=jnp.float32)


@functools.partial(jax.jit, static_argnames=("m", "nb"))
def _run(gxy, ls, xst, z, m, nb):
    b, _, n = xst.shape
    dz = z.shape[-1]
    return pl.pallas_call(
        _setconv_kernel,
        grid=(b // nb,),
        in_specs=[
            pl.BlockSpec((48, 2), lambda bb: (0, 0)),
            pl.BlockSpec((1, 2), lambda bb: (0, 0)),
            pl.BlockSpec((nb, 2, n), lambda bb: (bb, 0, 0)),
            pl.BlockSpec((nb, n, dz), lambda bb: (bb, 0, 0)),
        ],
        out_specs=pl.BlockSpec((nb, m, dz), lambda bb: (bb, 0, 0)),
        out_shape=jax.ShapeDtypeStruct((b, m, dz), jnp.float32),
        scratch_shapes=[pltpu.VMEM((_TM, n), jnp.float32)],
        compiler_params=pltpu.CompilerParams(
            dimension_semantics=("parallel",)),
    )(gxy, ls, xst, z)


def kernel(x, z, lengthscale_param):
    b, n, dx = x.shape
    dz = z.shape[-1]
    m = _POINTS[0] * _POINTS[1]
    gxy = jnp.asarray(_GXY)
    ls = lengthscale_param.reshape(1, 2)
    xst = x.transpose(0, 2, 1)
    z_grid_flat = _run(gxy, ls, xst, z, m=m, nb=_NB)
    z_grid = z_grid_flat.reshape((b,) + _POINTS + (dz,))
    x_grid = jnp.broadcast_to(jnp.asarray(_GRID)[None], (b,) + _GRID.shape)
    return (x_grid, z_grid)


# speedup vs baseline: 1.9054x; 1.0388x over previous
import functools

import jax
import jax.numpy as jnp
import numpy as np
from jax.experimental import pallas as pl
from jax.experimental.pallas import tpu as pltpu

_GRID_RANGE = ((-3.0, 3.0), (-3.0, 3.0))
_POINTS = (48, 48)
_TM = 576
_NB = 2


def _axes():
    return [np.linspace(lo, hi, p, dtype=np.float32)
            for (lo, hi), p in zip(_GRID_RANGE, _POINTS)]


_GRID = np.stack(np.meshgrid(*_axes(), indexing="ij"), axis=-1)
_GXY = np.stack(_axes(), axis=-1)


def _setconv_kernel(gxy_ref, ls_ref, xst_ref, z_ref, out_ref, w_ref):
    inv = 1.0 / (1e-05 + jax.nn.softplus(ls_ref[...]))
    gs = gxy_ref[...] * inv
    gx = gs[:, 0:1]
    gy = gs[:, 1:2]
    n_iy = gy.shape[0]
    rows_per_tile = _TM // n_iy
    n_tiles = (gx.shape[0] * n_iy) // _TM
    for bi in range(xst_ref.shape[0]):
        xs0 = xst_ref[bi, 0:1, :] * inv[:, 0:1]
        xs1 = xst_ref[bi, 1:2, :] * inv[:, 1:2]
        dx = gx - xs0
        dy = gy - xs1
        ex = jnp.exp(-0.5 * dx * dx)
        ey = jnp.exp(-0.5 * dy * dy)
        z = z_ref[bi]
        for g in range(n_tiles):
            for j in range(rows_per_tile):
                ix = g * rows_per_tile + j
                w_ref[j * n_iy:(j + 1) * n_iy, :] = ex[ix:ix + 1, :] * ey
            out_ref[bi, g * _TM:(g + 1) * _TM, :] = jnp.dot(
                w_ref[...], z, preferred_element_type=jnp.float32)


@functools.partial(jax.jit, static_argnames=("m", "nb"))
def _run(gxy, ls, xst, z, m, nb):
    b, _, n = xst.shape
    dz = z.shape[-1]
    return pl.pallas_call(
        _setconv_kernel,
        grid=(b // nb,),
        in_specs=[
            pl.BlockSpec((48, 2), lambda bb: (0, 0)),
            pl.BlockSpec((1, 2), lambda bb: (0, 0)),
            pl.BlockSpec((nb, 2, n), lambda bb: (bb, 0, 0)),
            pl.BlockSpec((nb, n, dz), lambda bb: (bb, 0, 0)),
        ],
        out_specs=pl.BlockSpec((nb, m, dz), lambda bb: (bb, 0, 0)),
        out_shape=jax.ShapeDtypeStruct((b, m, dz), jnp.float32),
        scratch_shapes=[pltpu.VMEM((_TM, n), jnp.float32)],
        compiler_params=pltpu.CompilerParams(
            dimension_semantics=("parallel",)),
    )(gxy, ls, xst, z)


def kernel(x, z, lengthscale_param):
    b, n, dx = x.shape
    dz = z.shape[-1]
    m = _POINTS[0] * _POINTS[1]
    gxy = jnp.asarray(_GXY)
    ls = lengthscale_param.reshape(1, 2)
    xst = x.transpose(0, 2, 1)
    z_grid_flat = _run(gxy, ls, xst, z, m=m, nb=_NB)
    z_grid = z_grid_flat.reshape((b,) + _POINTS + (dz,))
    x_grid = jnp.broadcast_to(jnp.asarray(_GRID)[None], (b,) + _GRID.shape)
    return (x_grid, z_grid)
